# baseline (speedup 1.0000x reference)
import jax
import jax.numpy as jnp
from jax.experimental import pallas as pl

_TM = 400
_TM_LIN = 2000


def _pick_tile(n, pref):
    for tm in (pref, 1000, 400, 200, 80, 40, 16, 8):
        if tm <= n and n % tm == 0:
            return tm
    return n


def _lin0_body(x_ref, w_ref, b_ref, o_ref):
    o_ref[...] = (
        jnp.dot(x_ref[...], w_ref[...], preferred_element_type=jnp.float32)
        + b_ref[...]
    )


def _layer1_body(adj_ref, g_ref, w_ref, b_ref, o_ref):
    h = jnp.dot(adj_ref[...], g_ref[...], preferred_element_type=jnp.float32)
    o_ref[...] = (
        jnp.dot(jnp.tanh(h), w_ref[...], preferred_element_type=jnp.float32)
        + b_ref[...]
    )


def _layer2_body(adj_ref, t_ref, o_ref):
    o_ref[...] = jnp.dot(adj_ref[...], t_ref[...], preferred_element_type=jnp.float32)


def kernel(x, adj, W0, b0, W1, b1):
    n, d_in = x.shape
    d_h = W0.shape[1]
    d_out = W1.shape[1]
    b0_2d = b0.reshape(1, d_h)
    b1_2d = b1.reshape(1, d_out)
    tm_lin = _pick_tile(n, _TM_LIN)
    tm = _pick_tile(n, _TM)

    g = pl.pallas_call(
        _lin0_body,
        grid=(n // tm_lin,),
        in_specs=[
            pl.BlockSpec((tm_lin, d_in), lambda i: (i, 0)),
            pl.BlockSpec((d_in, d_h), lambda i: (0, 0)),
            pl.BlockSpec((1, d_h), lambda i: (0, 0)),
        ],
        out_specs=pl.BlockSpec((tm_lin, d_h), lambda i: (i, 0)),
        out_shape=jax.ShapeDtypeStruct((n, d_h), jnp.float32),
    )(x, W0, b0_2d)

    t = pl.pallas_call(
        _layer1_body,
        grid=(n // tm,),
        in_specs=[
            pl.BlockSpec((tm, n), lambda i: (i, 0)),
            pl.BlockSpec((n, d_h), lambda i: (0, 0)),
            pl.BlockSpec((d_h, d_out), lambda i: (0, 0)),
            pl.BlockSpec((1, d_out), lambda i: (0, 0)),
        ],
        out_specs=pl.BlockSpec((tm, d_out), lambda i: (i, 0)),
        out_shape=jax.ShapeDtypeStruct((n, d_out), jnp.float32),
    )(adj, g, W1, b1_2d)

    z = pl.pallas_call(
        _layer2_body,
        grid=(n // tm,),
        in_specs=[
            pl.BlockSpec((tm, n), lambda i: (i, 0)),
            pl.BlockSpec((n, d_out), lambda i: (0, 0)),
        ],
        out_specs=pl.BlockSpec((tm, d_out), lambda i: (i, 0)),
        out_shape=jax.ShapeDtypeStruct((n, d_out), jnp.float32),
    )(adj, t)
    return z


# speedup vs baseline: 1.0551x; 1.0551x over previous
import jax
import jax.numpy as jnp
from jax.experimental import pallas as pl
from jax.experimental.pallas import tpu as pltpu

_TM = 400


def _pick_tile(n, pref):
    for tm in (pref, 1000, 400, 200, 80, 40, 16, 8):
        if tm <= n and n % tm == 0:
            return tm
    return n


def _gcn_body(x_ref, adj_ref, w0_ref, b0_ref, w1_ref, b1_ref, z_ref,
              g_scr, t_scr, *, tm):
    p = pl.program_id(0)
    i = pl.program_id(1)

    @pl.when((p == 0) & (i == 0))
    def _():
        g_scr[...] = (
            jnp.dot(x_ref[...], w0_ref[...], preferred_element_type=jnp.float32)
            + b0_ref[...]
        )

    @pl.when(p == 0)
    def _():
        h = jnp.dot(adj_ref[...], g_scr[...], preferred_element_type=jnp.float32)
        t_scr[pl.ds(i * tm, tm), :] = (
            jnp.dot(jnp.tanh(h), w1_ref[...], preferred_element_type=jnp.float32)
            + b1_ref[...]
        )
        z_ref[...] = jnp.zeros_like(z_ref)

    @pl.when(p == 1)
    def _():
        z_ref[...] = jnp.dot(
            adj_ref[...], t_scr[...], preferred_element_type=jnp.float32
        )


def kernel(x, adj, W0, b0, W1, b1):
    n, d_in = x.shape
    d_h = W0.shape[1]
    d_out = W1.shape[1]
    tm = _pick_tile(n, _TM)
    import functools

    body = functools.partial(_gcn_body, tm=tm)
    return pl.pallas_call(
        body,
        grid=(2, n // tm),
        in_specs=[
            pl.BlockSpec((n, d_in), lambda p, i: (0, 0)),
            pl.BlockSpec((tm, n), lambda p, i: (i, 0)),
            pl.BlockSpec((d_in, d_h), lambda p, i: (0, 0)),
            pl.BlockSpec((1, d_h), lambda p, i: (0, 0)),
            pl.BlockSpec((d_h, d_out), lambda p, i: (0, 0)),
            pl.BlockSpec((1, d_out), lambda p, i: (0, 0)),
        ],
        out_specs=pl.BlockSpec((tm, d_out), lambda p, i: (i, 0)),
        out_shape=jax.ShapeDtypeStruct((n, d_out), jnp.float32),
        scratch_shapes=[
            pltpu.VMEM((n, d_h), jnp.float32),
            pltpu.VMEM((n, d_out), jnp.float32),
        ],
    )(x, adj, W0, b0.reshape(1, d_h), W1, b1.reshape(1, d_out))
